# baseline (speedup 1.0000x reference)
import functools

import jax
import jax.numpy as jnp
from jax import lax
from jax.experimental import pallas as pl
from jax.experimental.pallas import tpu as pltpu
from jax.experimental.pallas import tpu_sc as plsc

_B, _P, _D, _OUT = 4096, 32, 128, 128

_NC = 2
_NS = 16
_NW = _NC * _NS
_NVR = _D // 16

_B_SC = 2048
_BLK = 256
_NSC_BLKS = _B_SC // _BLK
_NTC_BLKS = (_B - _B_SC) // _BLK


def _make_sc_reduce(b_sc, ch):
    epw = b_sc // _NW
    nch = epw // ch
    assert nch % 2 == 0

    def body(x_hbm, s_hbm, buf, acc, sem0, sem1):
        c = lax.axis_index("c")
        s = lax.axis_index("s")
        wid = s * _NC + c
        e0 = wid * epw
        sems = (sem0, sem1)

        def start(g, slot):
            return pltpu.async_copy(
                x_hbm.at[pl.ds(e0 + g * ch, ch)], buf.at[slot], sems[slot])

        def wait(g, slot):
            pltpu.make_async_copy(
                x_hbm.at[pl.ds(e0 + g * ch, ch)], buf.at[slot],
                sems[slot]).wait()

        def chunk_compute(slot, g):
            row0 = g * ch
            for e in range(ch):
                vs = [buf[slot, e, 0, pl.ds(16 * j, 16)]
                      for j in range(_NVR)]
                for p in range(1, _P):
                    for j in range(_NVR):
                        vs[j] = vs[j] + buf[slot, e, p, pl.ds(16 * j, 16)]
                for j in range(_NVR):
                    acc[row0 + e, pl.ds(16 * j, 16)] = vs[j]

        start(0, 0)
        start(1, 1)

        def gbody(g2, _):
            for slot in (0, 1):
                g = g2 * 2 + slot
                wait(g, slot)
                chunk_compute(slot, g)

                @pl.when(g + 2 < nch)
                def _():
                    start(g + 2, slot)
            return 0

        lax.fori_loop(0, nch // 2, gbody, 0)
        pltpu.sync_copy(acc, s_hbm.at[pl.ds(e0, epw)])

    return functools.partial(
        pl.kernel,
        body,
        out_type=jax.ShapeDtypeStruct((b_sc, _D), jnp.float32),
        scratch_types=[
            pltpu.VMEM((2, ch, _P, _D), jnp.float32),
            pltpu.VMEM((epw, _D), jnp.float32),
            pltpu.SemaphoreType.DMA,
            pltpu.SemaphoreType.DMA,
        ],
        mesh=plsc.VectorSubcoreMesh(core_axis_name="c", subcore_axis_name="s"),
    )()


_sc_reduce = _make_sc_reduce(_B_SC, 4)


def _fused_body(x_ref, w_ref, b_ref, o_ref):
    s = jnp.sum(x_ref[...], axis=1)
    y = jax.lax.dot_general(
        s, w_ref[...], (((1,), (1,)), ((), ())),
        preferred_element_type=jnp.float32,
        precision=jax.lax.Precision.HIGHEST,
    )
    o_ref[...] = y + float(_P) * b_ref[...]


def _tc_fused(x, W, b2d):
    return pl.pallas_call(
        _fused_body,
        grid=(_NTC_BLKS,),
        in_specs=[
            pl.BlockSpec((_BLK, _P, _D), lambda i: (i + _NSC_BLKS, 0, 0)),
            pl.BlockSpec((_OUT, _D), lambda i: (0, 0)),
            pl.BlockSpec((1, _OUT), lambda i: (0, 0)),
        ],
        out_specs=pl.BlockSpec((_BLK, _OUT), lambda i: (i + _NSC_BLKS, 0)),
        out_shape=jax.ShapeDtypeStruct((_B, _OUT), jnp.float32),
    )(x, W, b2d)


def _tail_body(sums_ref, w_ref, b_ref, part_ref, o_ref):
    o_ref[...] = jax.lax.dot_general(
        sums_ref[...], w_ref[...], (((1,), (1,)), ((), ())),
        preferred_element_type=jnp.float32,
        precision=jax.lax.Precision.HIGHEST,
    ) + float(_P) * b_ref[...]


def _tc_tail(sums, W, b2d, partial_out):
    tblk = _B_SC // 2
    return pl.pallas_call(
        _tail_body,
        grid=(2,),
        in_specs=[
            pl.BlockSpec((tblk, _D), lambda i: (i, 0)),
            pl.BlockSpec((_OUT, _D), lambda i: (0, 0)),
            pl.BlockSpec((1, _OUT), lambda i: (0, 0)),
            pl.BlockSpec(memory_space=pl.ANY),
        ],
        out_specs=pl.BlockSpec((tblk, _OUT), lambda i: (i, 0)),
        out_shape=jax.ShapeDtypeStruct((_B, _OUT), jnp.float32),
        input_output_aliases={3: 0},
    )(sums, W, b2d, partial_out)


def kernel(pert_batch, W, b):
    b2d = b.reshape(1, _OUT)
    sums = _sc_reduce(pert_batch)
    partial_out = _tc_fused(pert_batch, W, b2d)
    return _tc_tail(sums, W, b2d, partial_out)


# speedup vs baseline: 1.4456x; 1.4456x over previous
import functools

import jax
import jax.numpy as jnp
from jax import lax
from jax.experimental import pallas as pl
from jax.experimental.pallas import tpu as pltpu
from jax.experimental.pallas import tpu_sc as plsc

_B, _P, _D, _OUT = 4096, 32, 128, 128

_NC = 2
_NS = 16
_NW = _NC * _NS
_NVR = _D // 16

_B_SC = 1536
_BLK = 256
_NSC_BLKS = _B_SC // _BLK
_NTC_BLKS = (_B - _B_SC) // _BLK


def _make_sc_reduce(b_sc, ch):
    epw = b_sc // _NW
    nch = epw // ch
    assert nch % 2 == 0

    def body(x_hbm, s_hbm, buf, acc, sem0, sem1):
        c = lax.axis_index("c")
        s = lax.axis_index("s")
        wid = s * _NC + c
        e0 = wid * epw
        sems = (sem0, sem1)

        def start(g, slot):
            return pltpu.async_copy(
                x_hbm.at[pl.ds(e0 + g * ch, ch)], buf.at[slot], sems[slot])

        def wait(g, slot):
            pltpu.make_async_copy(
                x_hbm.at[pl.ds(e0 + g * ch, ch)], buf.at[slot],
                sems[slot]).wait()

        def chunk_compute(slot, g):
            row0 = g * ch
            for e in range(ch):
                def pbody(p, vs):
                    out = []
                    for j in range(_NVR):
                        v = vs[j]
                        for u in range(4):
                            v = v + buf[slot, e, p * 4 + u, pl.ds(16 * j, 16)]
                        out.append(v)
                    return tuple(out)

                vs0 = tuple(buf[slot, e, u, pl.ds(16 * j, 16)]
                            for j in range(_NVR) for u in (0,))
                vs0 = list(vs0)
                for j in range(_NVR):
                    for u in (1, 2, 3):
                        vs0[j] = vs0[j] + buf[slot, e, u, pl.ds(16 * j, 16)]
                vs = lax.fori_loop(1, _P // 4, pbody, tuple(vs0))
                for j in range(_NVR):
                    acc[row0 + e, pl.ds(16 * j, 16)] = vs[j]

        start(0, 0)
        start(1, 1)

        def gbody(g2, _):
            for slot in (0, 1):
                g = g2 * 2 + slot
                wait(g, slot)
                chunk_compute(slot, g)

                @pl.when(g + 2 < nch)
                def _():
                    start(g + 2, slot)
            return 0

        lax.fori_loop(0, nch // 2, gbody, 0)
        pltpu.sync_copy(acc, s_hbm.at[pl.ds(e0, epw)])

    return functools.partial(
        pl.kernel,
        body,
        out_type=jax.ShapeDtypeStruct((b_sc, _D), jnp.float32),
        scratch_types=[
            pltpu.VMEM((2, ch, _P, _D), jnp.float32),
            pltpu.VMEM((epw, _D), jnp.float32),
            pltpu.SemaphoreType.DMA,
            pltpu.SemaphoreType.DMA,
        ],
        mesh=plsc.VectorSubcoreMesh(core_axis_name="c", subcore_axis_name="s"),
    )()


_sc_reduce = _make_sc_reduce(_B_SC, 8)


def _fused_body(x_ref, w_ref, b_ref, o_ref):
    s = jnp.sum(x_ref[...], axis=1)
    y = jax.lax.dot_general(
        s, w_ref[...], (((1,), (1,)), ((), ())),
        preferred_element_type=jnp.float32,
        precision=jax.lax.Precision.HIGHEST,
    )
    o_ref[...] = y + float(_P) * b_ref[...]


def _tc_fused(x, W, b2d):
    return pl.pallas_call(
        _fused_body,
        grid=(_NTC_BLKS,),
        in_specs=[
            pl.BlockSpec((_BLK, _P, _D), lambda i: (i + _NSC_BLKS, 0, 0)),
            pl.BlockSpec((_OUT, _D), lambda i: (0, 0)),
            pl.BlockSpec((1, _OUT), lambda i: (0, 0)),
        ],
        out_specs=pl.BlockSpec((_BLK, _OUT), lambda i: (i + _NSC_BLKS, 0)),
        out_shape=jax.ShapeDtypeStruct((_B, _OUT), jnp.float32),
    )(x, W, b2d)


def _tail_body(sums_ref, w_ref, b_ref, part_ref, o_ref):
    o_ref[...] = jax.lax.dot_general(
        sums_ref[...], w_ref[...], (((1,), (1,)), ((), ())),
        preferred_element_type=jnp.float32,
        precision=jax.lax.Precision.HIGHEST,
    ) + float(_P) * b_ref[...]


def _tc_tail(sums, W, b2d, partial_out):
    tblk = _B_SC // 2
    return pl.pallas_call(
        _tail_body,
        grid=(2,),
        in_specs=[
            pl.BlockSpec((tblk, _D), lambda i: (i, 0)),
            pl.BlockSpec((_OUT, _D), lambda i: (0, 0)),
            pl.BlockSpec((1, _OUT), lambda i: (0, 0)),
            pl.BlockSpec(memory_space=pl.ANY),
        ],
        out_specs=pl.BlockSpec((tblk, _OUT), lambda i: (i, 0)),
        out_shape=jax.ShapeDtypeStruct((_B, _OUT), jnp.float32),
        input_output_aliases={3: 0},
    )(sums, W, b2d, partial_out)


def kernel(pert_batch, W, b):
    b2d = b.reshape(1, _OUT)
    sums = _sc_reduce(pert_batch)
    partial_out = _tc_fused(pert_batch, W, b2d)
    return _tc_tail(sums, W, b2d, partial_out)


# speedup vs baseline: 1.4491x; 1.0024x over previous
import functools

import jax
import jax.numpy as jnp
from jax import lax
from jax.experimental import pallas as pl
from jax.experimental.pallas import tpu as pltpu
from jax.experimental.pallas import tpu_sc as plsc

_B, _P, _D, _OUT = 4096, 32, 128, 128

_NC = 2
_NS = 16
_NW = _NC * _NS
_NVR = _D // 16

_B_SC = 1536
_BLK = 256
_NSC_BLKS = _B_SC // _BLK
_NTC_BLKS = (_B - _B_SC) // _BLK


def _make_sc_reduce(b_sc, ch):
    epw = b_sc // _NW
    nch = epw // ch
    assert nch % 2 == 0

    def body(x_hbm, s_hbm, buf, acc, sem0, sem1):
        c = lax.axis_index("c")
        s = lax.axis_index("s")
        wid = s * _NC + c
        e0 = wid * epw
        sems = (sem0, sem1)

        def start(g, slot):
            return pltpu.async_copy(
                x_hbm.at[pl.ds(e0 + g * ch, ch)], buf.at[slot], sems[slot])

        def wait(g, slot):
            pltpu.make_async_copy(
                x_hbm.at[pl.ds(e0 + g * ch, ch)], buf.at[slot],
                sems[slot]).wait()

        def chunk_compute(slot, g):
            row0 = g * ch

            def ebody(e, _):
                rows = buf.at[slot, e]
                vs = [rows[0, pl.ds(16 * j, 16)] for j in range(_NVR)]
                for p in range(1, _P):
                    for j in range(_NVR):
                        vs[j] = vs[j] + rows[p, pl.ds(16 * j, 16)]
                for j in range(_NVR):
                    acc[row0 + e, pl.ds(16 * j, 16)] = vs[j]
                return 0

            lax.fori_loop(0, ch, ebody, 0)

        start(0, 0)
        start(1, 1)

        def gbody(g2, _):
            for slot in (0, 1):
                g = g2 * 2 + slot
                wait(g, slot)
                chunk_compute(slot, g)

                @pl.when(g + 2 < nch)
                def _():
                    start(g + 2, slot)
            return 0

        lax.fori_loop(0, nch // 2, gbody, 0)
        pltpu.sync_copy(acc, s_hbm.at[pl.ds(e0, epw)])

    return functools.partial(
        pl.kernel,
        body,
        out_type=jax.ShapeDtypeStruct((b_sc, _D), jnp.float32),
        scratch_types=[
            pltpu.VMEM((2, ch, _P, _D), jnp.float32),
            pltpu.VMEM((epw, _D), jnp.float32),
            pltpu.SemaphoreType.DMA,
            pltpu.SemaphoreType.DMA,
        ],
        mesh=plsc.VectorSubcoreMesh(core_axis_name="c", subcore_axis_name="s"),
    )()


_sc_reduce = _make_sc_reduce(_B_SC, 8)


def _fused_body(x_ref, w_ref, b_ref, o_ref):
    s = jnp.sum(x_ref[...], axis=1)
    y = jax.lax.dot_general(
        s, w_ref[...], (((1,), (1,)), ((), ())),
        preferred_element_type=jnp.float32,
        precision=jax.lax.Precision.HIGHEST,
    )
    o_ref[...] = y + float(_P) * b_ref[...]


def _tc_fused(x, W, b2d):
    return pl.pallas_call(
        _fused_body,
        grid=(_NTC_BLKS,),
        in_specs=[
            pl.BlockSpec((_BLK, _P, _D), lambda i: (i + _NSC_BLKS, 0, 0)),
            pl.BlockSpec((_OUT, _D), lambda i: (0, 0)),
            pl.BlockSpec((1, _OUT), lambda i: (0, 0)),
        ],
        out_specs=pl.BlockSpec((_BLK, _OUT), lambda i: (i + _NSC_BLKS, 0)),
        out_shape=jax.ShapeDtypeStruct((_B, _OUT), jnp.float32),
    )(x, W, b2d)


def _tail_body(sums_ref, w_ref, b_ref, part_ref, o_ref):
    o_ref[...] = jax.lax.dot_general(
        sums_ref[...], w_ref[...], (((1,), (1,)), ((), ())),
        preferred_element_type=jnp.float32,
        precision=jax.lax.Precision.HIGHEST,
    ) + float(_P) * b_ref[...]


def _tc_tail(sums, W, b2d, partial_out):
    tblk = _B_SC // 2
    return pl.pallas_call(
        _tail_body,
        grid=(2,),
        in_specs=[
            pl.BlockSpec((tblk, _D), lambda i: (i, 0)),
            pl.BlockSpec((_OUT, _D), lambda i: (0, 0)),
            pl.BlockSpec((1, _OUT), lambda i: (0, 0)),
            pl.BlockSpec(memory_space=pl.ANY),
        ],
        out_specs=pl.BlockSpec((tblk, _OUT), lambda i: (i, 0)),
        out_shape=jax.ShapeDtypeStruct((_B, _OUT), jnp.float32),
        input_output_aliases={3: 0},
    )(sums, W, b2d, partial_out)


def kernel(pert_batch, W, b):
    b2d = b.reshape(1, _OUT)
    sums = _sc_reduce(pert_batch)
    partial_out = _tc_fused(pert_batch, W, b2d)
    return _tc_tail(sums, W, b2d, partial_out)


# speedup vs baseline: 1.5331x; 1.0580x over previous
import functools

import jax
import jax.numpy as jnp
from jax import lax
from jax.experimental import pallas as pl
from jax.experimental.pallas import tpu as pltpu
from jax.experimental.pallas import tpu_sc as plsc

_B, _P, _D, _OUT = 4096, 32, 128, 128

_NC = 2
_NS = 16
_NW = _NC * _NS
_NVR = _D // 16

_B_SC = 1280
_BLK = 256
_NSC_BLKS = _B_SC // _BLK
_NTC_BLKS = (_B - _B_SC) // _BLK


def _make_sc_reduce(b_sc, ch):
    epw = b_sc // _NW
    nch = epw // ch
    assert nch % 2 == 0

    def body(x_hbm, s_hbm, buf, acc, sem0, sem1):
        c = lax.axis_index("c")
        s = lax.axis_index("s")
        wid = s * _NC + c
        e0 = wid * epw
        sems = (sem0, sem1)

        def start(g, slot):
            return pltpu.async_copy(
                x_hbm.at[pl.ds(e0 + g * ch, ch)], buf.at[slot], sems[slot])

        def wait(g, slot):
            pltpu.make_async_copy(
                x_hbm.at[pl.ds(e0 + g * ch, ch)], buf.at[slot],
                sems[slot]).wait()

        def chunk_compute(slot, g):
            row0 = g * ch

            def ebody(e, _):
                rows = buf.at[slot, e]
                vs = [rows[0, pl.ds(16 * j, 16)] for j in range(_NVR)]
                for p in range(1, _P):
                    for j in range(_NVR):
                        vs[j] = vs[j] + rows[p, pl.ds(16 * j, 16)]
                for j in range(_NVR):
                    acc[row0 + e, pl.ds(16 * j, 16)] = vs[j]
                return 0

            lax.fori_loop(0, ch, ebody, 0)

        start(0, 0)
        start(1, 1)

        def gbody(g2, _):
            for slot in (0, 1):
                g = g2 * 2 + slot
                wait(g, slot)
                chunk_compute(slot, g)

                @pl.when(g + 2 < nch)
                def _():
                    start(g + 2, slot)
            return 0

        lax.fori_loop(0, nch // 2, gbody, 0)
        pltpu.sync_copy(acc, s_hbm.at[pl.ds(e0, epw)])

    return functools.partial(
        pl.kernel,
        body,
        out_type=jax.ShapeDtypeStruct((b_sc, _D), jnp.float32),
        scratch_types=[
            pltpu.VMEM((2, ch, _P, _D), jnp.float32),
            pltpu.VMEM((epw, _D), jnp.float32),
            pltpu.SemaphoreType.DMA,
            pltpu.SemaphoreType.DMA,
        ],
        mesh=plsc.VectorSubcoreMesh(core_axis_name="c", subcore_axis_name="s"),
    )()


_sc_reduce = _make_sc_reduce(_B_SC, 4)


def _fused_body(x_ref, w_ref, b_ref, o_ref):
    s = jnp.sum(x_ref[...], axis=1)
    y = jax.lax.dot_general(
        s, w_ref[...], (((1,), (1,)), ((), ())),
        preferred_element_type=jnp.float32,
        precision=jax.lax.Precision.HIGHEST,
    )
    o_ref[...] = y + float(_P) * b_ref[...]


def _tc_fused(x, W, b2d):
    return pl.pallas_call(
        _fused_body,
        grid=(_NTC_BLKS,),
        in_specs=[
            pl.BlockSpec((_BLK, _P, _D), lambda i: (i + _NSC_BLKS, 0, 0)),
            pl.BlockSpec((_OUT, _D), lambda i: (0, 0)),
            pl.BlockSpec((1, _OUT), lambda i: (0, 0)),
        ],
        out_specs=pl.BlockSpec((_BLK, _OUT), lambda i: (i + _NSC_BLKS, 0)),
        out_shape=jax.ShapeDtypeStruct((_B, _OUT), jnp.float32),
    )(x, W, b2d)


def _tail_body(sums_ref, w_ref, b_ref, part_ref, o_ref):
    o_ref[...] = jax.lax.dot_general(
        sums_ref[...], w_ref[...], (((1,), (1,)), ((), ())),
        preferred_element_type=jnp.float32,
        precision=jax.lax.Precision.HIGHEST,
    ) + float(_P) * b_ref[...]


def _tc_tail(sums, W, b2d, partial_out):
    tblk = _B_SC // 2
    return pl.pallas_call(
        _tail_body,
        grid=(2,),
        in_specs=[
            pl.BlockSpec((tblk, _D), lambda i: (i, 0)),
            pl.BlockSpec((_OUT, _D), lambda i: (0, 0)),
            pl.BlockSpec((1, _OUT), lambda i: (0, 0)),
            pl.BlockSpec(memory_space=pl.ANY),
        ],
        out_specs=pl.BlockSpec((tblk, _OUT), lambda i: (i, 0)),
        out_shape=jax.ShapeDtypeStruct((_B, _OUT), jnp.float32),
        input_output_aliases={3: 0},
    )(sums, W, b2d, partial_out)


def kernel(pert_batch, W, b):
    b2d = b.reshape(1, _OUT)
    sums = _sc_reduce(pert_batch)
    partial_out = _tc_fused(pert_batch, W, b2d)
    return _tc_tail(sums, W, b2d, partial_out)


# speedup vs baseline: 1.5699x; 1.0240x over previous
import functools

import jax
import jax.numpy as jnp
from jax import lax
from jax.experimental import pallas as pl
from jax.experimental.pallas import tpu as pltpu
from jax.experimental.pallas import tpu_sc as plsc

_B, _P, _D, _OUT = 4096, 32, 128, 128

_NC = 2
_NS = 16
_NW = _NC * _NS
_NVR = _D // 16

_B_SC = 1024
_BLK = 512
_NSC_BLKS = _B_SC // _BLK
_NTC_BLKS = (_B - _B_SC) // _BLK


def _make_sc_reduce(b_sc, ch):
    epw = b_sc // _NW
    nch = epw // ch
    assert nch % 2 == 0

    def body(x_hbm, s_hbm, buf, acc, sem0, sem1):
        c = lax.axis_index("c")
        s = lax.axis_index("s")
        wid = s * _NC + c
        e0 = wid * epw
        sems = (sem0, sem1)

        def start(g, slot):
            return pltpu.async_copy(
                x_hbm.at[pl.ds(e0 + g * ch, ch)], buf.at[slot], sems[slot])

        def wait(g, slot):
            pltpu.make_async_copy(
                x_hbm.at[pl.ds(e0 + g * ch, ch)], buf.at[slot],
                sems[slot]).wait()

        def chunk_compute(slot, g):
            row0 = g * ch

            def ebody(e, _):
                rows = buf.at[slot, e]
                vs = [rows[0, pl.ds(16 * j, 16)] for j in range(_NVR)]
                for p in range(1, _P):
                    for j in range(_NVR):
                        vs[j] = vs[j] + rows[p, pl.ds(16 * j, 16)]
                for j in range(_NVR):
                    acc[row0 + e, pl.ds(16 * j, 16)] = vs[j]
                return 0

            lax.fori_loop(0, ch, ebody, 0)

        start(0, 0)
        start(1, 1)

        def gbody(g2, _):
            for slot in (0, 1):
                g = g2 * 2 + slot
                wait(g, slot)
                chunk_compute(slot, g)

                @pl.when(g + 2 < nch)
                def _():
                    start(g + 2, slot)
            return 0

        lax.fori_loop(0, nch // 2, gbody, 0)
        pltpu.sync_copy(acc, s_hbm.at[pl.ds(e0, epw)])

    return functools.partial(
        pl.kernel,
        body,
        out_type=jax.ShapeDtypeStruct((b_sc, _D), jnp.float32),
        scratch_types=[
            pltpu.VMEM((2, ch, _P, _D), jnp.float32),
            pltpu.VMEM((epw, _D), jnp.float32),
            pltpu.SemaphoreType.DMA,
            pltpu.SemaphoreType.DMA,
        ],
        mesh=plsc.VectorSubcoreMesh(core_axis_name="c", subcore_axis_name="s"),
    )()


_sc_reduce = _make_sc_reduce(_B_SC, 4)


def _fused_body(x_ref, w_ref, b_ref, o_ref):
    s = jnp.sum(x_ref[...], axis=1)
    y = jax.lax.dot_general(
        s, w_ref[...], (((1,), (1,)), ((), ())),
        preferred_element_type=jnp.float32,
        precision=jax.lax.Precision.HIGHEST,
    )
    o_ref[...] = y + float(_P) * b_ref[...]


def _tc_fused(x, W, b2d):
    return pl.pallas_call(
        _fused_body,
        grid=(_NTC_BLKS,),
        in_specs=[
            pl.BlockSpec((_BLK, _P, _D), lambda i: (i + _NSC_BLKS, 0, 0)),
            pl.BlockSpec((_OUT, _D), lambda i: (0, 0)),
            pl.BlockSpec((1, _OUT), lambda i: (0, 0)),
        ],
        out_specs=pl.BlockSpec((_BLK, _OUT), lambda i: (i + _NSC_BLKS, 0)),
        out_shape=jax.ShapeDtypeStruct((_B, _OUT), jnp.float32),
    )(x, W, b2d)


def _tail_body(sums_ref, w_ref, b_ref, part_ref, o_ref):
    o_ref[...] = jax.lax.dot_general(
        sums_ref[...], w_ref[...], (((1,), (1,)), ((), ())),
        preferred_element_type=jnp.float32,
        precision=jax.lax.Precision.HIGHEST,
    ) + float(_P) * b_ref[...]


def _tc_tail(sums, W, b2d, partial_out):
    tblk = _B_SC // 2
    return pl.pallas_call(
        _tail_body,
        grid=(2,),
        in_specs=[
            pl.BlockSpec((tblk, _D), lambda i: (i, 0)),
            pl.BlockSpec((_OUT, _D), lambda i: (0, 0)),
            pl.BlockSpec((1, _OUT), lambda i: (0, 0)),
            pl.BlockSpec(memory_space=pl.ANY),
        ],
        out_specs=pl.BlockSpec((tblk, _OUT), lambda i: (i, 0)),
        out_shape=jax.ShapeDtypeStruct((_B, _OUT), jnp.float32),
        input_output_aliases={3: 0},
    )(sums, W, b2d, partial_out)


def kernel(pert_batch, W, b):
    b2d = b.reshape(1, _OUT)
    sums = _sc_reduce(pert_batch)
    partial_out = _tc_fused(pert_batch, W, b2d)
    return _tc_tail(sums, W, b2d, partial_out)
